# baseline (speedup 1.0000x reference)
import jax
import jax.numpy as jnp
from jax.experimental import pallas as pl
from jax.experimental.pallas import tpu as pltpu

_SHAPES = (
    (128, 80000, 1),
    (96, 80000, 1),
    (2080, 1250, 1),
    (512, 80000, 2),
    (1184, 80000, 4),
)
_NUM_DMAS = sum(c for _, _, c in _SHAPES)


def _copy_all_body(*refs):
    n = len(_SHAPES)
    ins, outs, sems = refs[:n], refs[n:2 * n], refs[2 * n]
    k = 0
    for (rows, _, chunks), x, o in zip(_SHAPES, ins, outs):
        step = rows // chunks
        for c in range(chunks):
            sl = pl.ds(c * step, step)
            pltpu.make_async_copy(x.at[sl], o.at[sl], sems.at[k]).start()
            k += 1
    k = 0
    for (rows, _, chunks), x, o in zip(_SHAPES, ins, outs):
        step = rows // chunks
        for c in range(chunks):
            sl = pl.ds(c * step, step)
            pltpu.make_async_copy(x.at[sl], o.at[sl], sems.at[k]).wait()
            k += 1


def _copy_all(tensors):
    flat = [t.reshape(r, c) for t, (r, c, _) in zip(tensors, _SHAPES)]
    outs = pl.pallas_call(
        _copy_all_body,
        in_specs=[pl.BlockSpec(memory_space=pl.ANY)] * len(flat),
        out_specs=[pl.BlockSpec(memory_space=pl.ANY)] * len(flat),
        out_shape=[jax.ShapeDtypeStruct(f.shape, f.dtype) for f in flat],
        scratch_shapes=[pltpu.SemaphoreType.DMA((_NUM_DMAS,))],
    )(*flat)
    return [o.reshape(t.shape) for o, t in zip(outs, tensors)]


def kernel(semantic, distance, vertex, embedding, direction):
    return tuple(_copy_all([semantic, distance, vertex, embedding, direction]))


# speedup vs baseline: 11.1217x; 11.1217x over previous
import jax
import jax.numpy as jnp
from jax.experimental import pallas as pl
from jax.experimental.pallas import tpu as pltpu

_WIDE = (
    (128, 80000),
    (96, 80000),
    (512, 80000),
    (1184, 80000),
)
_VERTEX = (2080, 1250)
_CHUNK_ROWS = 8
_NS = 16
_LAG = 8

_CHUNKS = [(t, r0) for t, (rows, _) in enumerate(_WIDE)
           for r0 in range(0, rows, _CHUNK_ROWS)]


def _stream_body(s0, s1, s2, s3, vx, o0, o1, o2, o3, ov,
                 ring, vbuf, rsem, wsem, vsem):
    ins = (s0, s1, s2, s3)
    outs = (o0, o1, o2, o3)
    n = len(_CHUNKS)

    def rd(i):
        t, r0 = _CHUNKS[i]
        return pltpu.make_async_copy(
            ins[t].at[pl.ds(r0, _CHUNK_ROWS)], ring.at[i % _NS], rsem.at[i % _NS])

    def wr(i):
        t, r0 = _CHUNKS[i]
        return pltpu.make_async_copy(
            ring.at[i % _NS], outs[t].at[pl.ds(r0, _CHUNK_ROWS)], wsem.at[i % _NS])

    v_in = pltpu.make_async_copy(vx, vbuf, vsem)
    v_out = pltpu.make_async_copy(vbuf, ov, vsem)

    v_in.start()
    for i in range(n + _LAG):
        if i < n:
            if i >= _NS:
                wr(i - _NS).wait()
            rd(i).start()
        j = i - _LAG
        if 0 <= j < n:
            rd(j).wait()
            wr(j).start()
        if i == 4:
            v_in.wait()
            v_out.start()
    for j in range(max(0, n - _NS), n):
        wr(j).wait()
    v_out.wait()


def kernel(semantic, distance, vertex, embedding, direction):
    wide = [semantic.reshape(_WIDE[0]), distance.reshape(_WIDE[1]),
            embedding.reshape(_WIDE[2]), direction.reshape(_WIDE[3])]
    vx = vertex.reshape(_VERTEX)
    outs = pl.pallas_call(
        _stream_body,
        in_specs=[pl.BlockSpec(memory_space=pl.ANY)] * 5,
        out_specs=[pl.BlockSpec(memory_space=pl.ANY)] * 5,
        out_shape=[jax.ShapeDtypeStruct(f.shape, f.dtype)
                   for f in (*wide, vx)],
        scratch_shapes=[
            pltpu.VMEM((_NS, _CHUNK_ROWS, 80000), jnp.float32),
            pltpu.VMEM(_VERTEX, jnp.float32),
            pltpu.SemaphoreType.DMA((_NS,)),
            pltpu.SemaphoreType.DMA((_NS,)),
            pltpu.SemaphoreType.DMA,
        ],
        compiler_params=pltpu.CompilerParams(vmem_limit_bytes=60 * 1024 * 1024),
    )(*wide, vx)
    return (outs[0].reshape(semantic.shape), outs[1].reshape(distance.shape),
            outs[4].reshape(vertex.shape), outs[2].reshape(embedding.shape),
            outs[3].reshape(direction.shape))


# speedup vs baseline: 36.2700x; 3.2612x over previous
import jax
import jax.numpy as jnp
from jax.experimental import pallas as pl
from jax.experimental.pallas import tpu as pltpu

_WIDE = (
    (25600, 400),
    (19200, 400),
    (102400, 400),
    (236800, 400),
)
_CHUNK_ROWS = 1600
_NS = 12
_LAG = 6

_VSHAPE = (2080, 25, 50)
_VCHUNK = 260
_VN = _VSHAPE[0] // _VCHUNK

_CHUNKS = [(t, r0) for t, (rows, _) in enumerate(_WIDE)
           for r0 in range(0, rows, _CHUNK_ROWS)]
_VSTEPS = {12 + 12 * k: k for k in range(_VN)}


def _stream_body(s0, s1, s2, s3, vx, o0, o1, o2, o3, ov,
                 ring, vring, rsem, wsem, vrsem, vwsem):
    ins = (s0, s1, s2, s3)
    outs = (o0, o1, o2, o3)
    n = len(_CHUNKS)

    def rd(i):
        t, r0 = _CHUNKS[i]
        return pltpu.make_async_copy(
            ins[t].at[pl.ds(r0, _CHUNK_ROWS)], ring.at[i % _NS], rsem.at[i % _NS])

    def wr(i):
        t, r0 = _CHUNKS[i]
        return pltpu.make_async_copy(
            ring.at[i % _NS], outs[t].at[pl.ds(r0, _CHUNK_ROWS)], wsem.at[i % _NS])

    def vrd(k):
        return pltpu.make_async_copy(
            vx.at[pl.ds(k * _VCHUNK, _VCHUNK)], vring.at[k % 2], vrsem.at[k % 2])

    def vwr(k):
        return pltpu.make_async_copy(
            vring.at[k % 2], ov.at[pl.ds(k * _VCHUNK, _VCHUNK)], vwsem.at[k % 2])

    vrd(0).start()
    vrd(1).start()
    for i in range(n + _LAG):
        if i < n:
            if i >= _NS:
                wr(i - _NS).wait()
            rd(i).start()
        j = i - _LAG
        if 0 <= j < n:
            rd(j).wait()
            wr(j).start()
        k = _VSTEPS.get(i)
        if k is not None:
            if k >= 2:
                vwr(k - 2).wait()
            vrd(k).wait()
            vwr(k).start()
            if k + 2 < _VN:
                vrd(k + 2).start()
    for j in range(n - _NS, n):
        wr(j).wait()
    vwr(_VN - 2).wait()
    vwr(_VN - 1).wait()


def kernel(semantic, distance, vertex, embedding, direction):
    wide = [semantic.reshape(_WIDE[0]), distance.reshape(_WIDE[1]),
            embedding.reshape(_WIDE[2]), direction.reshape(_WIDE[3])]
    vx = vertex.reshape(_VSHAPE)
    outs = pl.pallas_call(
        _stream_body,
        in_specs=[pl.BlockSpec(memory_space=pl.ANY)] * 5,
        out_specs=[pl.BlockSpec(memory_space=pl.ANY)] * 5,
        out_shape=[jax.ShapeDtypeStruct(f.shape, f.dtype)
                   for f in (*wide, vx)],
        scratch_shapes=[
            pltpu.VMEM((_NS, _CHUNK_ROWS, 400), jnp.float32),
            pltpu.VMEM((2, _VCHUNK) + _VSHAPE[1:], jnp.float32),
            pltpu.SemaphoreType.DMA((_NS,)),
            pltpu.SemaphoreType.DMA((_NS,)),
            pltpu.SemaphoreType.DMA((2,)),
            pltpu.SemaphoreType.DMA((2,)),
        ],
        compiler_params=pltpu.CompilerParams(vmem_limit_bytes=60 * 1024 * 1024),
    )(*wide, vx)
    return (outs[0].reshape(semantic.shape), outs[1].reshape(distance.shape),
            outs[4].reshape(vertex.shape), outs[2].reshape(embedding.shape),
            outs[3].reshape(direction.shape))
